# Initial kernel scaffold; baseline (speedup 1.0000x reference)
import jax
import jax.numpy as jnp
from jax.experimental import pallas as pl


def kernel(x, gate_W, Wk, bk, Wv, bv):
    raise NotImplementedError("write your pallas kernel here")


# speedup vs baseline: 1.4204x; 1.4204x over previous
import functools

import jax
import jax.numpy as jnp
from jax import lax
from jax.experimental import pallas as pl

D_MODEL = 768
HIDDEN = 2048
E = 64
T = 2048
K = 32


def _router_topk_body(x_ref, gw_ref, s_ref, i_ref):
    logits = jnp.dot(x_ref[...], gw_ref[...], preferred_element_type=jnp.float32)
    m = jnp.max(logits, axis=1, keepdims=True)
    p = jnp.exp(logits - m)
    probs = p / jnp.sum(p, axis=1, keepdims=True)
    iota0 = lax.broadcasted_iota(jnp.int32, (T, E), 0)
    vals = probs
    for i in range(K):
        mx = jnp.max(vals, axis=0, keepdims=True)
        am = jnp.min(jnp.where(vals == mx, iota0, T), axis=0, keepdims=True)
        s_ref[pl.ds(i, 1), :] = mx
        i_ref[pl.ds(i, 1), :] = am
        vals = jnp.where(iota0 == am, -jnp.inf, vals)


def _ffn_body(x_ref, s_ref, i_ref, wk_ref, bk_ref, wv_ref, bv_ref, out_ref):
    e = pl.program_id(0)
    idx_row = i_ref[0]
    sc_row = s_ref[0]
    iota_t = lax.broadcasted_iota(jnp.int32, (T, K), 0)
    onehot = (iota_t == idx_row).astype(jnp.float32)
    g = lax.dot_general(onehot, x_ref[...], (((0,), (0,)), ((), ())),
                        preferred_element_type=jnp.float32)
    h = jax.nn.gelu(jnp.dot(g, wk_ref[0], preferred_element_type=jnp.float32)
                    + bk_ref[0])
    o = jnp.dot(h, wv_ref[0], preferred_element_type=jnp.float32) + bv_ref[0]
    contrib = jnp.dot(onehot * sc_row, o, preferred_element_type=jnp.float32)

    @pl.when(e == 0)
    def _():
        out_ref[...] = contrib

    @pl.when(e != 0)
    def _():
        out_ref[...] += contrib


@jax.jit
def kernel(x, gate_W, Wk, bk, Wv, bv):
    b, s, d = x.shape
    x2d = x.reshape(T, D_MODEL)

    scores, idx = pl.pallas_call(
        _router_topk_body,
        out_shape=(
            jax.ShapeDtypeStruct((K, E), jnp.float32),
            jax.ShapeDtypeStruct((K, E), jnp.int32),
        ),
    )(x2d, gate_W)

    scores3 = scores.T.reshape(E, 1, K)
    idx3 = idx.T.reshape(E, 1, K)
    bk3 = bk.reshape(E, 1, HIDDEN)
    bv3 = bv.reshape(E, 1, D_MODEL)

    out = pl.pallas_call(
        _ffn_body,
        grid=(E,),
        in_specs=[
            pl.BlockSpec((T, D_MODEL), lambda e: (0, 0)),
            pl.BlockSpec((1, 1, K), lambda e: (e, 0, 0)),
            pl.BlockSpec((1, 1, K), lambda e: (e, 0, 0)),
            pl.BlockSpec((1, D_MODEL, HIDDEN), lambda e: (e, 0, 0)),
            pl.BlockSpec((1, 1, HIDDEN), lambda e: (e, 0, 0)),
            pl.BlockSpec((1, HIDDEN, D_MODEL), lambda e: (e, 0, 0)),
            pl.BlockSpec((1, 1, D_MODEL), lambda e: (e, 0, 0)),
        ],
        out_specs=pl.BlockSpec((T, D_MODEL), lambda e: (0, 0)),
        out_shape=jax.ShapeDtypeStruct((T, D_MODEL), jnp.float32),
    )(x2d, scores3, idx3, Wk, bk3, Wv, bv3)

    return out.reshape(b, s, d)


# speedup vs baseline: 1.4728x; 1.0368x over previous
import jax
import jax.numpy as jnp
from jax import lax
from jax.experimental import pallas as pl
from jax.experimental.pallas import tpu as pltpu

D_MODEL = 768
HIDDEN = 2048
E = 64
T = 2048
K = 32


def _moe_body(x_ref, gw_ref, wk_ref, bk_ref, wv_ref, bv_ref, out_ref,
              s_scr, i_scr):
    e = pl.program_id(0)

    @pl.when(e == 0)
    def _router_topk():
        logits = jnp.dot(x_ref[...], gw_ref[...],
                         preferred_element_type=jnp.float32)
        m = jnp.max(logits, axis=1, keepdims=True)
        p = jnp.exp(logits - m)
        probs = p / jnp.sum(p, axis=1, keepdims=True)
        vals = jnp.transpose(probs)
        iota1 = lax.broadcasted_iota(jnp.int32, (E, T), 1)
        for i in range(K):
            mx = jnp.max(vals, axis=1, keepdims=True)
            am = jnp.min(jnp.where(vals == mx, iota1, T), axis=1, keepdims=True)
            s_scr[:, pl.ds(i, 1)] = mx
            i_scr[:, pl.ds(i, 1)] = am
            vals = jnp.where(iota1 == am, -jnp.inf, vals)

    idx_row = i_scr[pl.ds(e, 1), :]
    sc_row = s_scr[pl.ds(e, 1), :]
    iota_t = lax.broadcasted_iota(jnp.int32, (T, K), 0)
    onehot = (iota_t == idx_row).astype(jnp.float32)
    g = lax.dot_general(onehot, x_ref[...], (((0,), (0,)), ((), ())),
                        preferred_element_type=jnp.float32)
    h = jax.nn.gelu(jnp.dot(g, wk_ref[0], preferred_element_type=jnp.float32)
                    + bk_ref[0])
    o = jnp.dot(h, wv_ref[0], preferred_element_type=jnp.float32) + bv_ref[0]
    contrib = jnp.dot(onehot * sc_row, o, preferred_element_type=jnp.float32)

    @pl.when(e == 0)
    def _():
        out_ref[...] = contrib

    @pl.when(e != 0)
    def _():
        out_ref[...] += contrib


@jax.jit
def kernel(x, gate_W, Wk, bk, Wv, bv):
    b, s, d = x.shape
    x2d = x.reshape(T, D_MODEL)
    bk3 = bk.reshape(E, 1, HIDDEN)
    bv3 = bv.reshape(E, 1, D_MODEL)

    out = pl.pallas_call(
        _moe_body,
        grid=(E,),
        in_specs=[
            pl.BlockSpec((T, D_MODEL), lambda e: (0, 0)),
            pl.BlockSpec((D_MODEL, E), lambda e: (0, 0)),
            pl.BlockSpec((1, D_MODEL, HIDDEN), lambda e: (e, 0, 0)),
            pl.BlockSpec((1, 1, HIDDEN), lambda e: (e, 0, 0)),
            pl.BlockSpec((1, HIDDEN, D_MODEL), lambda e: (e, 0, 0)),
            pl.BlockSpec((1, 1, D_MODEL), lambda e: (e, 0, 0)),
        ],
        out_specs=pl.BlockSpec((T, D_MODEL), lambda e: (0, 0)),
        out_shape=jax.ShapeDtypeStruct((T, D_MODEL), jnp.float32),
        scratch_shapes=[
            pltpu.VMEM((E, K), jnp.float32),
            pltpu.VMEM((E, K), jnp.int32),
        ],
    )(x2d, gate_W, Wk, bk3, Wv, bv3)

    return out.reshape(b, s, d)


# speedup vs baseline: 1.5909x; 1.0802x over previous
import jax
import jax.numpy as jnp
from jax import lax
from jax.experimental import pallas as pl
from jax.experimental.pallas import tpu as pltpu

D_MODEL = 768
HIDDEN = 2048
E = 64
T = 2048
K = 32
G = 8
NBLK = E // G
S = G * K


def _moe_body(x_ref, gw_ref, wk_ref, bk_ref, wv_ref, bv_ref, out_ref,
              s_scr, i_scr, sf_scr, if_scr, oacc_scr):
    e = pl.program_id(0)

    @pl.when(e == 0)
    def _router_topk():
        logits = jnp.dot(x_ref[...], gw_ref[...],
                         preferred_element_type=jnp.float32)
        m = jnp.max(logits, axis=1, keepdims=True)
        p = jnp.exp(logits - m)
        probs = p / jnp.sum(p, axis=1, keepdims=True)
        vals = jnp.transpose(probs)
        iota1 = lax.broadcasted_iota(jnp.int32, (E, T), 1)
        for i in range(K):
            mx = jnp.max(vals, axis=1, keepdims=True)
            am = jnp.min(jnp.where(vals == mx, iota1, T), axis=1, keepdims=True)
            s_scr[:, pl.ds(i, 1)] = mx
            i_scr[:, pl.ds(i, 1)] = am
            vals = jnp.where(iota1 == am, -jnp.inf, vals)
        for ee in range(E):
            sf_scr[ee // G, :, pl.ds((ee % G) * K, K)] = s_scr[pl.ds(ee, 1), :]
            if_scr[ee // G, :, pl.ds((ee % G) * K, K)] = i_scr[pl.ds(ee, 1), :]

    idx_row = i_scr[pl.ds(e, 1), :]
    iota_t = lax.broadcasted_iota(jnp.int32, (T, K), 0)
    onehot = (iota_t == idx_row).astype(jnp.float32)
    g = lax.dot_general(onehot, x_ref[...], (((0,), (0,)), ((), ())),
                        preferred_element_type=jnp.float32)
    h = jax.nn.gelu(jnp.dot(g, wk_ref[0], preferred_element_type=jnp.float32)
                    + bk_ref[0])
    o = jnp.dot(h, wv_ref[0], preferred_element_type=jnp.float32) + bv_ref[0]
    oacc_scr[pl.ds(lax.rem(e, G) * K, K), :] = o

    @pl.when(lax.rem(e, G) == G - 1)
    def _flush():
        b = lax.div(e, G)
        idx_flat = if_scr[pl.ds(b, 1)].reshape(1, S)
        sc_flat = sf_scr[pl.ds(b, 1)].reshape(1, S)
        iota_s = lax.broadcasted_iota(jnp.int32, (T, S), 0)
        oh = (iota_s == idx_flat).astype(jnp.float32) * sc_flat
        contrib = jnp.dot(oh, oacc_scr[...], preferred_element_type=jnp.float32)

        @pl.when(b == 0)
        def _():
            out_ref[...] = contrib

        @pl.when(b != 0)
        def _():
            out_ref[...] += contrib


@jax.jit
def kernel(x, gate_W, Wk, bk, Wv, bv):
    b, s, d = x.shape
    x2d = x.reshape(T, D_MODEL)
    bk3 = bk.reshape(E, 1, HIDDEN)
    bv3 = bv.reshape(E, 1, D_MODEL)

    out = pl.pallas_call(
        _moe_body,
        grid=(E,),
        in_specs=[
            pl.BlockSpec((T, D_MODEL), lambda e: (0, 0)),
            pl.BlockSpec((D_MODEL, E), lambda e: (0, 0)),
            pl.BlockSpec((1, D_MODEL, HIDDEN), lambda e: (e, 0, 0)),
            pl.BlockSpec((1, 1, HIDDEN), lambda e: (e, 0, 0)),
            pl.BlockSpec((1, HIDDEN, D_MODEL), lambda e: (e, 0, 0)),
            pl.BlockSpec((1, 1, D_MODEL), lambda e: (e, 0, 0)),
        ],
        out_specs=pl.BlockSpec((T, D_MODEL), lambda e: (0, 0)),
        out_shape=jax.ShapeDtypeStruct((T, D_MODEL), jnp.float32),
        scratch_shapes=[
            pltpu.VMEM((E, K), jnp.float32),
            pltpu.VMEM((E, K), jnp.int32),
            pltpu.VMEM((NBLK, 1, S), jnp.float32),
            pltpu.VMEM((NBLK, 1, S), jnp.int32),
            pltpu.VMEM((S, D_MODEL), jnp.float32),
        ],
    )(x2d, gate_W, Wk, bk3, Wv, bv3)

    return out.reshape(b, s, d)
